# baseline (speedup 1.0000x reference)
import functools

import jax
import jax.numpy as jnp
from jax import lax
from jax.experimental import pallas as pl
from jax.experimental.pallas import tpu as pltpu
from jax.experimental.pallas import tpu_sc as plsc

_TOP_K = 96
_LANES = 16
_NUM_WORKERS = 32
_UNROLL = 16


def _reduce_row(buf, hw, acc_ref, row_local):
    chunk = _UNROLL * _LANES
    n_outer = hw // chunk

    def body(j, acc):
        base = j * chunk
        vs = [buf[pl.ds(base + u * _LANES, _LANES)] for u in range(_UNROLL)]
        m = [jnp.abs(v) for v in vs]
        while len(m) > 1:
            m = [jnp.maximum(m[i], m[i + 1]) for i in range(0, len(m) - 1, 2)] + (
                [m[-1]] if len(m) % 2 else []
            )
        return jnp.maximum(acc, m[0])

    acc = lax.fori_loop(0, n_outer, body, jnp.zeros((_LANES,), jnp.float32))
    acc_ref[pl.ds(row_local * _LANES, _LANES)] = acc


def _sc_score_kernel(rows, hw):
    rpw = rows // _NUM_WORKERS
    mesh = plsc.VectorSubcoreMesh(core_axis_name="c", subcore_axis_name="s")

    @functools.partial(
        pl.kernel,
        mesh=mesh,
        out_type=jax.ShapeDtypeStruct((rows * _LANES,), jnp.float32),
        scratch_types=[
            pltpu.VMEM((hw,), jnp.float32),
            pltpu.VMEM((hw,), jnp.float32),
            pltpu.VMEM((rpw * _LANES,), jnp.float32),
            pltpu.SemaphoreType.DMA,
            pltpu.SemaphoreType.DMA,
        ],
    )
    def k(x_hbm, out_hbm, buf0, buf1, acc, sem0, sem1):
        wid = lax.axis_index("s") * 2 + lax.axis_index("c")
        base = wid * rpw

        pltpu.make_async_copy(x_hbm.at[base], buf0, sem0).start()

        def outer(i, _):
            r0 = base + i * 2
            pltpu.make_async_copy(x_hbm.at[r0 + 1], buf1, sem1).start()
            pltpu.make_async_copy(x_hbm.at[r0], buf0, sem0).wait()
            _reduce_row(buf0, hw, acc, i * 2)

            @pl.when(i * 2 + 2 < rpw)
            def _():
                pltpu.make_async_copy(x_hbm.at[r0 + 2], buf0, sem0).start()

            pltpu.make_async_copy(x_hbm.at[r0 + 1], buf1, sem1).wait()
            _reduce_row(buf1, hw, acc, i * 2 + 1)
            return 0

        lax.fori_loop(0, rpw // 2, outer, 0)
        pltpu.sync_copy(acc, out_hbm.at[pl.ds(base * _LANES, rpw * _LANES)])

    return k


def _topk_body(k: int, s_ref, o_ref, s2_ref):
    s2_ref[...] = jnp.max(s_ref[...], axis=2)
    s = s2_ref[...]
    b, c = s.shape
    si = s[:, :, None]
    sj = s[:, None, :]
    ii = lax.broadcasted_iota(jnp.int32, (b, c, c), 1)
    jj = lax.broadcasted_iota(jnp.int32, (b, c, c), 2)
    beats = (sj > si) | ((sj == si) & (jj < ii))
    rank = jnp.sum(beats.astype(jnp.int32), axis=2)
    pos = lax.broadcasted_iota(jnp.int32, (b, c, k), 2)
    chan = lax.broadcasted_iota(jnp.int32, (b, c, k), 1)
    hit = rank[:, :, None] == pos
    o_ref[...] = jnp.sum(jnp.where(hit, chan, 0), axis=1)


_WINDOW = 64


def _gather_body(idx_ref, x_hbm, o_hbm, sem):
    b, k = idx_ref.shape

    def copy_at(t):
        bi = t // k
        ki = t % k
        ch = idx_ref[bi, ki]
        return pltpu.make_async_copy(x_hbm.at[bi, ch], o_hbm.at[bi, ki], sem)

    def start_loop(t, _):
        copy_at(t).start()

        @pl.when(t >= _WINDOW)
        def _():
            copy_at(t - _WINDOW).wait()

        return 0

    total = b * k
    lax.fori_loop(0, total, start_loop, 0)

    def drain(t, _):
        copy_at(total - _WINDOW + t).wait()
        return 0

    lax.fori_loop(0, _WINDOW, drain, 0)


def _channel_topk_pool(x, k: int):
    b, c, h, w = x.shape
    hw = h * w
    rows = b * c
    x2 = x.reshape(rows, hw)

    scores16 = _sc_score_kernel(rows, hw)(x2)
    s3 = scores16.reshape(b, c, _LANES)

    idx = pl.pallas_call(
        functools.partial(_topk_body, k),
        in_specs=[pl.BlockSpec((b, c, _LANES), lambda: (0, 0, 0))],
        out_specs=pl.BlockSpec((b, k), lambda: (0, 0)),
        out_shape=jax.ShapeDtypeStruct((b, k), jnp.int32),
        scratch_shapes=[pltpu.VMEM((b, c), jnp.float32)],
    )(s3)

    out = pl.pallas_call(
        _gather_body,
        grid_spec=pltpu.PrefetchScalarGridSpec(
            num_scalar_prefetch=1,
            grid=(1,),
            in_specs=[pl.BlockSpec(memory_space=pltpu.MemorySpace.HBM)],
            out_specs=pl.BlockSpec(memory_space=pltpu.MemorySpace.HBM),
            scratch_shapes=[pltpu.SemaphoreType.DMA],
        ),
        out_shape=jax.ShapeDtypeStruct((b, k, h, w), jnp.float32),
    )(idx, x)
    return out


def kernel(x):
    return _channel_topk_pool(x, _TOP_K)


# speedup vs baseline: 2.3533x; 2.3533x over previous
import functools

import jax
import jax.numpy as jnp
from jax import lax
from jax.experimental import pallas as pl
from jax.experimental.pallas import tpu as pltpu
from jax.experimental.pallas import tpu_sc as plsc

_TOP_K = 96
_LANES = 16
_NUM_WORKERS = 32
_UNROLL = 16


def _reduce_row(buf, hw, acc_ref, row_local):
    chunk = _UNROLL * _LANES
    n_outer = hw // chunk

    def body(j, acc):
        base = j * chunk
        vs = [buf[pl.ds(base + u * _LANES, _LANES)] for u in range(_UNROLL)]
        m = [jnp.abs(v) for v in vs]
        while len(m) > 1:
            m = [jnp.maximum(m[i], m[i + 1]) for i in range(0, len(m) - 1, 2)] + (
                [m[-1]] if len(m) % 2 else []
            )
        return jnp.maximum(acc, m[0])

    acc = lax.fori_loop(0, n_outer, body, jnp.zeros((_LANES,), jnp.float32))
    acc_ref[pl.ds(row_local * _LANES, _LANES)] = acc


def _sc_score_kernel(rows, hw):
    rpw = rows // _NUM_WORKERS
    mesh = plsc.VectorSubcoreMesh(core_axis_name="c", subcore_axis_name="s")

    @functools.partial(
        pl.kernel,
        mesh=mesh,
        out_type=jax.ShapeDtypeStruct((rows * _LANES,), jnp.float32),
        scratch_types=[
            pltpu.VMEM((hw,), jnp.float32),
            pltpu.VMEM((hw,), jnp.float32),
            pltpu.VMEM((rpw * _LANES,), jnp.float32),
            pltpu.SemaphoreType.DMA,
            pltpu.SemaphoreType.DMA,
        ],
    )
    def k(x_hbm, out_hbm, buf0, buf1, acc, sem0, sem1):
        wid = lax.axis_index("s") * 2 + lax.axis_index("c")
        base = wid * rpw

        pltpu.make_async_copy(x_hbm.at[base], buf0, sem0).start()

        def outer(i, _):
            r0 = base + i * 2
            pltpu.make_async_copy(x_hbm.at[r0 + 1], buf1, sem1).start()
            pltpu.make_async_copy(x_hbm.at[r0], buf0, sem0).wait()
            _reduce_row(buf0, hw, acc, i * 2)

            @pl.when(i * 2 + 2 < rpw)
            def _():
                pltpu.make_async_copy(x_hbm.at[r0 + 2], buf0, sem0).start()

            pltpu.make_async_copy(x_hbm.at[r0 + 1], buf1, sem1).wait()
            _reduce_row(buf1, hw, acc, i * 2 + 1)
            return 0

        lax.fori_loop(0, rpw // 2, outer, 0)
        pltpu.sync_copy(acc, out_hbm.at[pl.ds(base * _LANES, rpw * _LANES)])

    return k


def _topk_body(k: int, s_ref, o_ref, s2_ref):
    s2_ref[...] = jnp.max(s_ref[...], axis=2)
    s = s2_ref[...]
    b, c = s.shape
    si = s[:, :, None]
    sj = s[:, None, :]
    ii = lax.broadcasted_iota(jnp.int32, (b, c, c), 1)
    jj = lax.broadcasted_iota(jnp.int32, (b, c, c), 2)
    beats = (sj > si) | ((sj == si) & (jj < ii))
    rank = jnp.sum(beats.astype(jnp.int32), axis=2)
    pos = lax.broadcasted_iota(jnp.int32, (b, c, k), 2)
    chan = lax.broadcasted_iota(jnp.int32, (b, c, k), 1)
    hit = rank[:, :, None] == pos
    o_ref[...] = jnp.sum(jnp.where(hit, chan, 0), axis=1)


_GATHER_CHUNK = 8


def _gather_body(idx_ref, *refs):
    del idx_ref
    xs = refs[:-1]
    o_ref = refs[-1]
    for j, x_ref in enumerate(xs):
        o_ref[0, j] = x_ref[0]


def _channel_topk_pool(x, k: int):
    b, c, h, w = x.shape
    hw = h * w
    rows = b * c
    x2 = x.reshape(rows, hw)

    scores16 = _sc_score_kernel(rows, hw)(x2)
    s3 = scores16.reshape(b, c, _LANES)

    idx = pl.pallas_call(
        functools.partial(_topk_body, k),
        in_specs=[pl.BlockSpec((b, c, _LANES), lambda: (0, 0, 0))],
        out_specs=pl.BlockSpec((b, k), lambda: (0, 0)),
        out_shape=jax.ShapeDtypeStruct((b, k), jnp.int32),
        scratch_shapes=[pltpu.VMEM((b, c), jnp.float32)],
    )(s3)

    g = _GATHER_CHUNK

    if hw % 128 == 0:
        m, n = hw // 128, 128
    else:
        m, n = h, w
    x3 = x2.reshape(rows, m, n)

    def _in_spec(j):
        return pl.BlockSpec(
            (1, m, n), lambda bi, ki, idx_r: (bi * c + idx_r[bi, ki * g + j], 0, 0)
        )

    out3 = pl.pallas_call(
        _gather_body,
        grid_spec=pltpu.PrefetchScalarGridSpec(
            num_scalar_prefetch=1,
            grid=(b, k // g),
            in_specs=[_in_spec(j) for j in range(g)],
            out_specs=pl.BlockSpec(
                (1, g, m, n), lambda bi, ki, idx_r: (bi, ki, 0, 0)
            ),
        ),
        out_shape=jax.ShapeDtypeStruct((b, k, m, n), jnp.float32),
    )(idx, *([x3] * g))
    return out3.reshape(b, k, h, w)


def kernel(x):
    return _channel_topk_pool(x, _TOP_K)


# speedup vs baseline: 6.7320x; 2.8606x over previous
import functools

import jax
import jax.numpy as jnp
from jax import lax
from jax.experimental import pallas as pl
from jax.experimental.pallas import tpu as pltpu
from jax.experimental.pallas import tpu_sc as plsc

_TOP_K = 96
_LANES = 16
_NUM_WORKERS = 32
_UNROLL = 16


def _reduce_row(buf, h, w, acc_ref, row_local):
    nvec = w // _LANES

    def body(i, acc):
        vs = [buf[i, pl.ds(u * _LANES, _LANES)] for u in range(nvec)]
        m = [jnp.abs(v) for v in vs]
        while len(m) > 1:
            m = [jnp.maximum(m[i2], m[i2 + 1]) for i2 in range(0, len(m) - 1, 2)] + (
                [m[-1]] if len(m) % 2 else []
            )
        return jnp.maximum(acc, m[0])

    acc = lax.fori_loop(0, h, body, jnp.zeros((_LANES,), jnp.float32))
    acc_ref[pl.ds(row_local * _LANES, _LANES)] = acc


def _sc_score_kernel(rows, h, w):
    rpw = rows // _NUM_WORKERS
    mesh = plsc.VectorSubcoreMesh(core_axis_name="c", subcore_axis_name="s")

    @functools.partial(
        pl.kernel,
        mesh=mesh,
        out_type=jax.ShapeDtypeStruct((rows * _LANES,), jnp.float32),
        scratch_types=[
            pltpu.VMEM((h, w), jnp.float32),
            pltpu.VMEM((h, w), jnp.float32),
            pltpu.VMEM((rpw * _LANES,), jnp.float32),
            pltpu.SemaphoreType.DMA,
            pltpu.SemaphoreType.DMA,
        ],
    )
    def k(x_hbm, out_hbm, buf0, buf1, acc, sem0, sem1):
        wid = lax.axis_index("s") * 2 + lax.axis_index("c")
        base = wid * rpw

        pltpu.make_async_copy(x_hbm.at[base], buf0, sem0).start()

        def outer(i, _):
            r0 = base + i * 2
            pltpu.make_async_copy(x_hbm.at[r0 + 1], buf1, sem1).start()
            pltpu.make_async_copy(x_hbm.at[r0], buf0, sem0).wait()
            _reduce_row(buf0, h, w, acc, i * 2)

            @pl.when(i * 2 + 2 < rpw)
            def _():
                pltpu.make_async_copy(x_hbm.at[r0 + 2], buf0, sem0).start()

            pltpu.make_async_copy(x_hbm.at[r0 + 1], buf1, sem1).wait()
            _reduce_row(buf1, h, w, acc, i * 2 + 1)
            return 0

        lax.fori_loop(0, rpw // 2, outer, 0)
        pltpu.sync_copy(acc, out_hbm.at[pl.ds(base * _LANES, rpw * _LANES)])

    return k


def _topk_body(k: int, s_ref, o_ref, s2_ref):
    s2_ref[...] = jnp.max(s_ref[...], axis=2)
    s = s2_ref[...]
    b, c = s.shape
    si = s[:, :, None]
    sj = s[:, None, :]
    ii = lax.broadcasted_iota(jnp.int32, (b, c, c), 1)
    jj = lax.broadcasted_iota(jnp.int32, (b, c, c), 2)
    beats = (sj > si) | ((sj == si) & (jj < ii))
    rank = jnp.sum(beats.astype(jnp.int32), axis=2)
    pos = lax.broadcasted_iota(jnp.int32, (b, c, k), 2)
    chan = lax.broadcasted_iota(jnp.int32, (b, c, k), 1)
    hit = rank[:, :, None] == pos
    o_ref[...] = jnp.sum(jnp.where(hit, chan, 0), axis=1)


_GATHER_CHUNK = 8


def _gather_body(idx_ref, *refs):
    del idx_ref
    xs = refs[:-1]
    o_ref = refs[-1]
    for j, x_ref in enumerate(xs):
        o_ref[0, j] = x_ref[0, 0]


def _channel_topk_pool(x, k: int):
    b, c, h, w = x.shape
    rows = b * c
    x3 = x.reshape(rows, h, w)

    scores16 = _sc_score_kernel(rows, h, w)(x3)
    s3 = scores16.reshape(b, c, _LANES)

    idx = pl.pallas_call(
        functools.partial(_topk_body, k),
        in_specs=[pl.BlockSpec((b, c, _LANES), lambda: (0, 0, 0))],
        out_specs=pl.BlockSpec((b, k), lambda: (0, 0)),
        out_shape=jax.ShapeDtypeStruct((b, k), jnp.int32),
        scratch_shapes=[pltpu.VMEM((b, c), jnp.float32)],
    )(s3)

    g = _GATHER_CHUNK

    def _in_spec(j):
        return pl.BlockSpec(
            (1, 1, h, w),
            lambda bi, ki, idx_r: (bi, idx_r[bi, ki * g + j], 0, 0),
        )

    out = pl.pallas_call(
        _gather_body,
        grid_spec=pltpu.PrefetchScalarGridSpec(
            num_scalar_prefetch=1,
            grid=(b, k // g),
            in_specs=[_in_spec(j) for j in range(g)],
            out_specs=pl.BlockSpec(
                (1, g, h, w), lambda bi, ki, idx_r: (bi, ki, 0, 0)
            ),
        ),
        out_shape=jax.ShapeDtypeStruct((b, k, h, w), jnp.float32),
    )(idx, *([x] * g))
    return out


def kernel(x):
    return _channel_topk_pool(x, _TOP_K)


# speedup vs baseline: 6.8301x; 1.0146x over previous
import functools

import jax
import jax.numpy as jnp
from jax import lax
from jax.experimental import pallas as pl
from jax.experimental.pallas import tpu as pltpu
from jax.experimental.pallas import tpu_sc as plsc

_TOP_K = 96
_LANES = 16
_NUM_WORKERS = 32
_UNROLL = 16


def _reduce_row(buf, h, w, acc_ref, row_local):
    nvec = w // _LANES

    def body(i, acc):
        vs = [buf[i, pl.ds(u * _LANES, _LANES)] for u in range(nvec)]
        m = [jnp.abs(v) for v in vs]
        while len(m) > 1:
            m = [jnp.maximum(m[i2], m[i2 + 1]) for i2 in range(0, len(m) - 1, 2)] + (
                [m[-1]] if len(m) % 2 else []
            )
        return jnp.maximum(acc, m[0])

    acc = lax.fori_loop(0, h, body, jnp.zeros((_LANES,), jnp.float32))
    acc_ref[pl.ds(row_local * _LANES, _LANES)] = acc


def _sc_score_kernel(rows, h, w):
    rpw = rows // _NUM_WORKERS
    mesh = plsc.VectorSubcoreMesh(core_axis_name="c", subcore_axis_name="s")

    @functools.partial(
        pl.kernel,
        mesh=mesh,
        out_type=jax.ShapeDtypeStruct((rows * _LANES,), jnp.float32),
        scratch_types=[
            pltpu.VMEM((h, w), jnp.float32),
            pltpu.VMEM((h, w), jnp.float32),
            pltpu.VMEM((rpw * _LANES,), jnp.float32),
            pltpu.SemaphoreType.DMA,
            pltpu.SemaphoreType.DMA,
        ],
    )
    def k(x_hbm, out_hbm, buf0, buf1, acc, sem0, sem1):
        wid = lax.axis_index("s") * 2 + lax.axis_index("c")
        base = wid * rpw

        pltpu.make_async_copy(x_hbm.at[base], buf0, sem0).start()

        def outer(i, _):
            r0 = base + i * 2
            pltpu.make_async_copy(x_hbm.at[r0 + 1], buf1, sem1).start()
            pltpu.make_async_copy(x_hbm.at[r0], buf0, sem0).wait()
            _reduce_row(buf0, h, w, acc, i * 2)

            @pl.when(i * 2 + 2 < rpw)
            def _():
                pltpu.make_async_copy(x_hbm.at[r0 + 2], buf0, sem0).start()

            pltpu.make_async_copy(x_hbm.at[r0 + 1], buf1, sem1).wait()
            _reduce_row(buf1, h, w, acc, i * 2 + 1)
            return 0

        lax.fori_loop(0, rpw // 2, outer, 0)
        pltpu.sync_copy(acc, out_hbm.at[pl.ds(base * _LANES, rpw * _LANES)])

    return k


def _topk_body(k: int, s_ref, o_ref, s2_ref):
    s2_ref[...] = jnp.max(s_ref[...], axis=2)
    s = s2_ref[...]
    b, c = s.shape
    si = s[:, :, None]
    sj = s[:, None, :]
    ii = lax.broadcasted_iota(jnp.int32, (b, c, c), 1)
    jj = lax.broadcasted_iota(jnp.int32, (b, c, c), 2)
    beats = (sj > si) | ((sj == si) & (jj < ii))
    rank = jnp.sum(beats.astype(jnp.int32), axis=2)
    pos = lax.broadcasted_iota(jnp.int32, (b, c, k), 2)
    chan = lax.broadcasted_iota(jnp.int32, (b, c, k), 1)
    hit = rank[:, :, None] == pos
    o_ref[...] = jnp.sum(jnp.where(hit, chan, 0), axis=1)


_GATHER_CHUNK = 16


def _gather_body(idx_ref, *refs):
    del idx_ref
    xs = refs[:-1]
    o_ref = refs[-1]
    for j, x_ref in enumerate(xs):
        o_ref[0, j] = x_ref[0, 0]


def _channel_topk_pool(x, k: int):
    b, c, h, w = x.shape
    rows = b * c
    x3 = x.reshape(rows, h, w)

    scores16 = _sc_score_kernel(rows, h, w)(x3)
    s3 = scores16.reshape(b, c, _LANES)

    idx = pl.pallas_call(
        functools.partial(_topk_body, k),
        in_specs=[pl.BlockSpec((b, c, _LANES), lambda: (0, 0, 0))],
        out_specs=pl.BlockSpec((b, k), lambda: (0, 0)),
        out_shape=jax.ShapeDtypeStruct((b, k), jnp.int32),
        scratch_shapes=[pltpu.VMEM((b, c), jnp.float32)],
    )(s3)

    g = _GATHER_CHUNK

    def _in_spec(j):
        return pl.BlockSpec(
            (1, 1, h, w),
            lambda bi, ki, idx_r: (bi, idx_r[bi, ki * g + j], 0, 0),
        )

    out = pl.pallas_call(
        _gather_body,
        grid_spec=pltpu.PrefetchScalarGridSpec(
            num_scalar_prefetch=1,
            grid=(b, k // g),
            in_specs=[_in_spec(j) for j in range(g)],
            out_specs=pl.BlockSpec(
                (1, g, h, w), lambda bi, ki, idx_r: (bi, ki, 0, 0)
            ),
        ),
        out_shape=jax.ShapeDtypeStruct((b, k, h, w), jnp.float32),
    )(idx, *([x] * g))
    return out


def kernel(x):
    return _channel_topk_pool(x, _TOP_K)


# speedup vs baseline: 7.2399x; 1.0600x over previous
import jax
import jax.numpy as jnp
from jax import lax
from jax.experimental import pallas as pl
from jax.experimental.pallas import tpu as pltpu

_TOP_K = 96
_SCORE_BLK = 32
_GATHER_CHUNK = 16


def _score_body(x_ref, o_ref):
    o_ref[pl.program_id(0), :] = jnp.max(jnp.abs(x_ref[...]), axis=(1, 2))


def _topk_body(k: int, s_ref, o_ref):
    s = s_ref[...]
    b, c = s.shape
    si = s[:, :, None]
    sj = s[:, None, :]
    ii = lax.broadcasted_iota(jnp.int32, (b, c, c), 1)
    jj = lax.broadcasted_iota(jnp.int32, (b, c, c), 2)
    beats = (sj > si) | ((sj == si) & (jj < ii))
    rank = jnp.sum(beats.astype(jnp.int32), axis=2)
    pos = lax.broadcasted_iota(jnp.int32, (b, c, k), 2)
    chan = lax.broadcasted_iota(jnp.int32, (b, c, k), 1)
    hit = rank[:, :, None] == pos
    o_ref[...] = jnp.sum(jnp.where(hit, chan, 0), axis=1)


def _gather_body(idx_ref, *refs):
    del idx_ref
    xs = refs[:-1]
    o_ref = refs[-1]
    for j, x_ref in enumerate(xs):
        o_ref[0, j] = x_ref[0, 0]


def _channel_topk_pool(x, k: int):
    b, c, h, w = x.shape
    rows = b * c
    x3 = x.reshape(rows, h, w)

    blk = _SCORE_BLK
    scores2 = pl.pallas_call(
        _score_body,
        grid=(rows // blk,),
        in_specs=[pl.BlockSpec((blk, h, w), lambda i: (i, 0, 0))],
        out_specs=pl.BlockSpec((rows // blk, blk), lambda i: (0, 0)),
        out_shape=jax.ShapeDtypeStruct((rows // blk, blk), jnp.float32),
    )(x3)
    scores = scores2.reshape(b, c)

    idx = pl.pallas_call(
        lambda s_ref, o_ref: _topk_body(k, s_ref, o_ref),
        in_specs=[pl.BlockSpec((b, c), lambda: (0, 0))],
        out_specs=pl.BlockSpec((b, k), lambda: (0, 0)),
        out_shape=jax.ShapeDtypeStruct((b, k), jnp.int32),
    )(scores)

    g = _GATHER_CHUNK

    def _in_spec(j):
        return pl.BlockSpec(
            (1, 1, h, w),
            lambda bi, ki, idx_r: (bi, idx_r[bi, ki * g + j], 0, 0),
        )

    out = pl.pallas_call(
        _gather_body,
        grid_spec=pltpu.PrefetchScalarGridSpec(
            num_scalar_prefetch=1,
            grid=(b, k // g),
            in_specs=[_in_spec(j) for j in range(g)],
            out_specs=pl.BlockSpec(
                (1, g, h, w), lambda bi, ki, idx_r: (bi, ki, 0, 0)
            ),
        ),
        out_shape=jax.ShapeDtypeStruct((b, k, h, w), jnp.float32),
    )(idx, *([x] * g))
    return out


def kernel(x):
    return _channel_topk_pool(x, _TOP_K)


# speedup vs baseline: 7.2499x; 1.0014x over previous
import jax
import jax.numpy as jnp
from jax import lax
from jax.experimental import pallas as pl
from jax.experimental.pallas import tpu as pltpu

_TOP_K = 96
_GATHER_CHUNK = 16


_SCORE_STREAMS = 8
_SCORE_ROWS = 8


def _score_body(*refs):
    xs = refs[:-1]
    o_ref = refs[-1]
    i = pl.program_id(0)
    rb = _SCORE_ROWS
    for j, x_ref in enumerate(xs):
        o_ref[i, j * rb : (j + 1) * rb] = jnp.max(jnp.abs(x_ref[...]), axis=(1, 2))


def _topk_body(k: int, s_ref, o_ref):
    s = s_ref[...]
    b, c = s.shape
    si = s[:, :, None]
    sj = s[:, None, :]
    ii = lax.broadcasted_iota(jnp.int32, (b, c, c), 1)
    jj = lax.broadcasted_iota(jnp.int32, (b, c, c), 2)
    beats = (sj > si) | ((sj == si) & (jj < ii))
    rank = jnp.sum(beats.astype(jnp.int32), axis=2)
    pos = lax.broadcasted_iota(jnp.int32, (b, c, k), 2)
    chan = lax.broadcasted_iota(jnp.int32, (b, c, k), 1)
    hit = rank[:, :, None] == pos
    o_ref[...] = jnp.sum(jnp.where(hit, chan, 0), axis=1)


def _gather_body(idx_ref, *refs):
    del idx_ref
    xs = refs[:-1]
    o_ref = refs[-1]
    for j, x_ref in enumerate(xs):
        o_ref[0, j] = x_ref[0, 0]


def _channel_topk_pool(x, k: int):
    b, c, h, w = x.shape
    rows = b * c
    x3 = x.reshape(rows, h, w)

    ns, rb = _SCORE_STREAMS, _SCORE_ROWS
    step_rows = ns * rb
    grid_n = rows // step_rows

    def _score_in_spec(j):
        return pl.BlockSpec((rb, h, w), lambda i: (i * ns + j, 0, 0))

    scores2 = pl.pallas_call(
        _score_body,
        grid=(grid_n,),
        in_specs=[_score_in_spec(j) for j in range(ns)],
        out_specs=pl.BlockSpec((grid_n, step_rows), lambda i: (0, 0)),
        out_shape=jax.ShapeDtypeStruct((grid_n, step_rows), jnp.float32),
    )(*([x3] * ns))
    scores = scores2.reshape(b, c)

    idx = pl.pallas_call(
        lambda s_ref, o_ref: _topk_body(k, s_ref, o_ref),
        in_specs=[pl.BlockSpec((b, c), lambda: (0, 0))],
        out_specs=pl.BlockSpec((b, k), lambda: (0, 0)),
        out_shape=jax.ShapeDtypeStruct((b, k), jnp.int32),
    )(scores)

    g = _GATHER_CHUNK

    def _in_spec(j):
        return pl.BlockSpec(
            (1, 1, h, w),
            lambda bi, ki, idx_r: (bi, idx_r[bi, ki * g + j], 0, 0),
        )

    out = pl.pallas_call(
        _gather_body,
        grid_spec=pltpu.PrefetchScalarGridSpec(
            num_scalar_prefetch=1,
            grid=(b, k // g),
            in_specs=[_in_spec(j) for j in range(g)],
            out_specs=pl.BlockSpec(
                (1, g, h, w), lambda bi, ki, idx_r: (bi, ki, 0, 0)
            ),
        ),
        out_shape=jax.ShapeDtypeStruct((b, k, h, w), jnp.float32),
    )(idx, *([x] * g))
    return out


def kernel(x):
    return _channel_topk_pool(x, _TOP_K)
